# baseline (speedup 1.0000x reference)
import dataclasses
import functools

import jax
import jax.numpy as jnp
from jax import lax
from jax.experimental import pallas as pl
from jax.experimental.pallas import tpu as pltpu
from jax.experimental.pallas import tpu_sc as plsc

LANDMARK_GENES = 978
VOCAB_SIZE = 20000
HIDDEN_SIZE = 128
BATCH = 1024

LANES = 16
N_WORKERS = 32
CHUNK = 512
N_SPLIT = 4
SUB = CHUNK // N_SPLIT
N_REAL = LANDMARK_GENES * BATCH // CHUNK
CHUNKS_PER_WORKER = -(-N_REAL // N_WORKERS)
N_PAD = N_WORKERS * CHUNKS_PER_WORKER
PER_WORKER = CHUNKS_PER_WORKER * CHUNK


def _scale_rows(rows_vmem, gex_vmem, row_lo, gex_lo, n_rows):
    off = gex_lo - row_lo

    @plsc.parallel_loop(row_lo, row_lo + n_rows, unroll=16)
    def _(r):
        gidx = jnp.full((LANES,), r + off, jnp.int32)
        g = plsc.load_gather(gex_vmem, [gidx])
        for c in range(HIDDEN_SIZE // LANES):
            sl = (r, pl.ds(c * LANES, LANES))
            rows_vmem[sl] = rows_vmem[sl] * g


def _gex_embed(ids_flat, gex_flat, table):
    mesh = plsc.VectorSubcoreMesh(core_axis_name="c", subcore_axis_name="s")
    cp = pltpu.CompilerParams()
    for _field, _val in (("needs_layout_passes", False),
                         ("use_tc_tiling_on_sc", False)):
        if _field in pltpu.CompilerParams.__dataclass_fields__:
            cp = dataclasses.replace(cp, **{_field: _val})

    @functools.partial(
        pl.kernel,
        out_type=jax.ShapeDtypeStruct(
            (LANDMARK_GENES, BATCH, HIDDEN_SIZE), jnp.float32
        ),
        mesh=mesh,
        compiler_params=cp,
        scratch_types=(
            [pltpu.VMEM((PER_WORKER,), jnp.int32),
             pltpu.VMEM((PER_WORKER,), jnp.float32),
             pltpu.VMEM((CHUNK, HIDDEN_SIZE), jnp.float32)]
            + [pltpu.SemaphoreType.DMA] * (2 + 2 * N_SPLIT)
        ),
    )
    def k(table_hbm, ids_hbm, gex_hbm, out_hbm, ids_v, gex_v, rows_v, *sems):
        sem_ids, sem_gex = sems[0], sems[1]
        gsem = sems[2:2 + N_SPLIT]
        osem = sems[2 + N_SPLIT:]
        wid = lax.axis_index("s") * 2 + lax.axis_index("c")
        row0 = wid * PER_WORKER

        stage_i = pltpu.async_copy(
            ids_hbm.at[pl.ds(row0, PER_WORKER)], ids_v, sem_ids)
        stage_g = pltpu.async_copy(
            gex_hbm.at[pl.ds(row0, PER_WORKER)], gex_v, sem_gex)
        stage_i.wait()
        stage_g.wait()

        @pl.loop(0, CHUNKS_PER_WORKER)
        def _(kk):
            ci = wid * CHUNKS_PER_WORKER + kk
            real = ci < N_REAL
            gene = ci // 2
            half = lax.rem(ci, 2)
            base = kk * CHUNK

            for q in range(N_SPLIT):
                @pl.when(jnp.logical_and(kk > 0, real))
                def _():
                    pltpu.make_async_copy(
                        rows_v.at[pl.ds(q * SUB, SUB)],
                        out_hbm.at[0, pl.ds(0, SUB)],
                        osem[q],
                    ).wait()

            @pl.when(real)
            def _():
                copies = []
                for q in range(N_SPLIT):
                    copies.append(pltpu.async_copy(
                        table_hbm.at[ids_v.at[pl.ds(base + q * SUB, SUB)]],
                        rows_v.at[pl.ds(q * SUB, SUB)],
                        gsem[q],
                    ))
                for q in range(N_SPLIT):
                    copies[q].wait()
                    _scale_rows(rows_v, gex_v, q * SUB, base + q * SUB, SUB)
                    pltpu.async_copy(
                        rows_v.at[pl.ds(q * SUB, SUB)],
                        out_hbm.at[gene, pl.ds(half * CHUNK + q * SUB, SUB)],
                        osem[q],
                    )

        for q in range(N_SPLIT):
            pltpu.make_async_copy(
                rows_v.at[pl.ds(q * SUB, SUB)],
                out_hbm.at[0, pl.ds(0, SUB)],
                osem[q],
            ).wait()

    return k(table, ids_flat, gex_flat)


def kernel(gene_expression, gene_input_ids, bool_masked_pos, group_mtx, gene_embedding_table):
    del bool_masked_pos, group_mtx
    pad = N_PAD * CHUNK - LANDMARK_GENES * BATCH
    ids_flat = jnp.pad(
        gene_input_ids.astype(jnp.int32).T.reshape(-1), (0, pad))
    gex_flat = jnp.pad(
        gene_expression.astype(jnp.float32).T.reshape(-1), (0, pad))
    out_t = _gex_embed(ids_flat, gex_flat, gene_embedding_table)
    return jnp.transpose(out_t, (1, 0, 2))


# speedup vs baseline: 1.1392x; 1.1392x over previous
import dataclasses
import functools

import jax
import jax.numpy as jnp
from jax import lax
from jax.experimental import pallas as pl
from jax.experimental.pallas import tpu as pltpu
from jax.experimental.pallas import tpu_sc as plsc

LANDMARK_GENES = 978
VOCAB_SIZE = 20000
HIDDEN_SIZE = 128
BATCH = 1024

LANES = 16
N_WORKERS = 32
CHUNK = 512
N_SPLIT = 4
SUB = CHUNK // N_SPLIT
N_REAL = LANDMARK_GENES * BATCH // CHUNK
CHUNKS_PER_WORKER = -(-N_REAL // N_WORKERS)
N_PAD = N_WORKERS * CHUNKS_PER_WORKER
PER_WORKER = CHUNKS_PER_WORKER * CHUNK


def _scale_rows(rows_vmem, gex_vmem, row_lo, gex_lo, n_rows):
    off = gex_lo - row_lo

    @plsc.parallel_loop(row_lo, row_lo + n_rows, unroll=16)
    def _(r):
        gidx = jnp.full((LANES,), r + off, jnp.int32)
        g = plsc.load_gather(gex_vmem, [gidx])
        for c in range(HIDDEN_SIZE // LANES):
            sl = (r, pl.ds(c * LANES, LANES))
            rows_vmem[sl] = rows_vmem[sl] * g


def _gex_embed(ids_flat, gex_flat, table):
    mesh = plsc.VectorSubcoreMesh(core_axis_name="c", subcore_axis_name="s")
    cp = pltpu.CompilerParams()
    for _field, _val in (("needs_layout_passes", False),
                         ("use_tc_tiling_on_sc", False)):
        if _field in pltpu.CompilerParams.__dataclass_fields__:
            cp = dataclasses.replace(cp, **{_field: _val})

    @functools.partial(
        pl.kernel,
        out_type=jax.ShapeDtypeStruct(
            (LANDMARK_GENES, BATCH, HIDDEN_SIZE), jnp.float32
        ),
        mesh=mesh,
        compiler_params=cp,
        scratch_types=(
            [pltpu.VMEM((PER_WORKER,), jnp.int32),
             pltpu.VMEM((PER_WORKER,), jnp.float32),
             pltpu.VMEM((CHUNK, HIDDEN_SIZE), jnp.float32)]
            + [pltpu.SemaphoreType.DMA] * (2 + 2 * N_SPLIT)
        ),
    )
    def k(table_hbm, ids_hbm, gex_hbm, out_hbm, ids_v, gex_v, rows_v, *sems):
        sem_ids, sem_gex = sems[0], sems[1]
        gsem = sems[2:2 + N_SPLIT]
        osem = sems[2 + N_SPLIT:]
        wid = lax.axis_index("s") * 2 + lax.axis_index("c")
        row0 = wid * PER_WORKER

        stage_i = pltpu.async_copy(
            ids_hbm.at[pl.ds(row0, PER_WORKER)], ids_v, sem_ids)
        stage_g = pltpu.async_copy(
            gex_hbm.at[pl.ds(row0, PER_WORKER)], gex_v, sem_gex)
        stage_i.wait()
        stage_g.wait()

        @pl.loop(0, CHUNKS_PER_WORKER)
        def _(kk):
            ci = wid * CHUNKS_PER_WORKER + kk
            real = ci < N_REAL
            gene = ci // 2
            half = lax.rem(ci, 2)
            base = kk * CHUNK

            @pl.when(real)
            def _():
                copies = []
                for q in range(N_SPLIT):
                    @pl.when(kk > 0)
                    def _():
                        pltpu.make_async_copy(
                            rows_v.at[pl.ds(q * SUB, SUB)],
                            out_hbm.at[0, pl.ds(0, SUB)],
                            osem[q],
                        ).wait()

                    copies.append(pltpu.async_copy(
                        table_hbm.at[ids_v.at[pl.ds(base + q * SUB, SUB)]],
                        rows_v.at[pl.ds(q * SUB, SUB)],
                        gsem[q],
                    ))
                for q in range(N_SPLIT):
                    copies[q].wait()
                    _scale_rows(rows_v, gex_v, q * SUB, base + q * SUB, SUB)
                    pltpu.async_copy(
                        rows_v.at[pl.ds(q * SUB, SUB)],
                        out_hbm.at[gene, pl.ds(half * CHUNK + q * SUB, SUB)],
                        osem[q],
                    )

        for q in range(N_SPLIT):
            pltpu.make_async_copy(
                rows_v.at[pl.ds(q * SUB, SUB)],
                out_hbm.at[0, pl.ds(0, SUB)],
                osem[q],
            ).wait()

    return k(table, ids_flat, gex_flat)


def kernel(gene_expression, gene_input_ids, bool_masked_pos, group_mtx, gene_embedding_table):
    del bool_masked_pos, group_mtx
    pad = N_PAD * CHUNK - LANDMARK_GENES * BATCH
    ids_flat = jnp.pad(
        gene_input_ids.astype(jnp.int32).T.reshape(-1), (0, pad))
    gex_flat = jnp.pad(
        gene_expression.astype(jnp.float32).T.reshape(-1), (0, pad))
    out_t = _gex_embed(ids_flat, gex_flat, gene_embedding_table)
    return jnp.transpose(out_t, (1, 0, 2))


# speedup vs baseline: 1.1884x; 1.0432x over previous
import dataclasses
import functools

import jax
import jax.numpy as jnp
from jax import lax
from jax.experimental import pallas as pl
from jax.experimental.pallas import tpu as pltpu
from jax.experimental.pallas import tpu_sc as plsc

LANDMARK_GENES = 978
VOCAB_SIZE = 20000
HIDDEN_SIZE = 128
BATCH = 1024

LANES = 16
N_WORKERS = 32
CHUNK = 512
N_SPLIT = 8
SUB = CHUNK // N_SPLIT
N_REAL = LANDMARK_GENES * BATCH // CHUNK
CHUNKS_PER_WORKER = -(-N_REAL // N_WORKERS)
N_PAD = N_WORKERS * CHUNKS_PER_WORKER
PER_WORKER = CHUNKS_PER_WORKER * CHUNK


def _scale_rows(rows_vmem, gex_vmem, row_lo, gex_lo, n_rows):
    off = gex_lo - row_lo

    @plsc.parallel_loop(row_lo, row_lo + n_rows, unroll=16)
    def _(r):
        gidx = jnp.full((LANES,), r + off, jnp.int32)
        g = plsc.load_gather(gex_vmem, [gidx])
        for c in range(HIDDEN_SIZE // LANES):
            sl = (r, pl.ds(c * LANES, LANES))
            rows_vmem[sl] = rows_vmem[sl] * g


def _gex_embed(ids_flat, gex_flat, table):
    mesh = plsc.VectorSubcoreMesh(core_axis_name="c", subcore_axis_name="s")
    cp = pltpu.CompilerParams()
    for _field, _val in (("needs_layout_passes", False),
                         ("use_tc_tiling_on_sc", False)):
        if _field in pltpu.CompilerParams.__dataclass_fields__:
            cp = dataclasses.replace(cp, **{_field: _val})

    @functools.partial(
        pl.kernel,
        out_type=jax.ShapeDtypeStruct(
            (LANDMARK_GENES, BATCH, HIDDEN_SIZE), jnp.float32
        ),
        mesh=mesh,
        compiler_params=cp,
        scratch_types=(
            [pltpu.VMEM((PER_WORKER,), jnp.int32),
             pltpu.VMEM((PER_WORKER,), jnp.float32),
             pltpu.VMEM((CHUNK, HIDDEN_SIZE), jnp.float32)]
            + [pltpu.SemaphoreType.DMA] * (2 + 2 * N_SPLIT)
        ),
    )
    def k(table_hbm, ids_hbm, gex_hbm, out_hbm, ids_v, gex_v, rows_v, *sems):
        sem_ids, sem_gex = sems[0], sems[1]
        gsem = sems[2:2 + N_SPLIT]
        osem = sems[2 + N_SPLIT:]
        wid = lax.axis_index("s") * 2 + lax.axis_index("c")
        row0 = wid * PER_WORKER

        stage_i = pltpu.async_copy(
            ids_hbm.at[pl.ds(row0, PER_WORKER)], ids_v, sem_ids)
        stage_g = pltpu.async_copy(
            gex_hbm.at[pl.ds(row0, PER_WORKER)], gex_v, sem_gex)
        stage_i.wait()
        stage_g.wait()

        @pl.loop(0, CHUNKS_PER_WORKER)
        def _(kk):
            ci = wid * CHUNKS_PER_WORKER + kk
            real = ci < N_REAL
            gene = ci // 2
            half = lax.rem(ci, 2)
            base = kk * CHUNK

            @pl.when(real)
            def _():
                copies = []
                for q in range(N_SPLIT):
                    @pl.when(kk > 0)
                    def _():
                        pltpu.make_async_copy(
                            rows_v.at[pl.ds(q * SUB, SUB)],
                            out_hbm.at[0, pl.ds(0, SUB)],
                            osem[q],
                        ).wait()

                    copies.append(pltpu.async_copy(
                        table_hbm.at[ids_v.at[pl.ds(base + q * SUB, SUB)]],
                        rows_v.at[pl.ds(q * SUB, SUB)],
                        gsem[q],
                    ))
                for q in range(N_SPLIT):
                    copies[q].wait()
                    _scale_rows(rows_v, gex_v, q * SUB, base + q * SUB, SUB)
                    pltpu.async_copy(
                        rows_v.at[pl.ds(q * SUB, SUB)],
                        out_hbm.at[gene, pl.ds(half * CHUNK + q * SUB, SUB)],
                        osem[q],
                    )

        for q in range(N_SPLIT):
            pltpu.make_async_copy(
                rows_v.at[pl.ds(q * SUB, SUB)],
                out_hbm.at[0, pl.ds(0, SUB)],
                osem[q],
            ).wait()

    return k(table, ids_flat, gex_flat)


def kernel(gene_expression, gene_input_ids, bool_masked_pos, group_mtx, gene_embedding_table):
    del bool_masked_pos, group_mtx
    pad = N_PAD * CHUNK - LANDMARK_GENES * BATCH
    ids_flat = jnp.pad(
        gene_input_ids.astype(jnp.int32).T.reshape(-1), (0, pad))
    gex_flat = jnp.pad(
        gene_expression.astype(jnp.float32).T.reshape(-1), (0, pad))
    out_t = _gex_embed(ids_flat, gex_flat, gene_embedding_table)
    return jnp.transpose(out_t, (1, 0, 2))


# speedup vs baseline: 1.2270x; 1.0324x over previous
import dataclasses
import functools

import jax
import jax.numpy as jnp
from jax import lax
from jax.experimental import pallas as pl
from jax.experimental.pallas import tpu as pltpu
from jax.experimental.pallas import tpu_sc as plsc

LANDMARK_GENES = 978
VOCAB_SIZE = 20000
HIDDEN_SIZE = 128
BATCH = 1024

LANES = 16
N_WORKERS = 32
SUB = 64
N_SLOTS = 8
LEAD = 4
N_ROWS = LANDMARK_GENES * BATCH
N_STEPS = N_ROWS // (SUB * N_WORKERS)
PER_WORKER = N_STEPS * SUB


def _scale_rows(rows_vmem, gex_vmem, row_lo, gex_lo):
    off = gex_lo - row_lo

    @plsc.parallel_loop(row_lo, row_lo + SUB, unroll=16)
    def _(r):
        gidx = jnp.full((LANES,), r + off, jnp.int32)
        g = plsc.load_gather(gex_vmem, [gidx])
        for c in range(HIDDEN_SIZE // LANES):
            sl = (r, pl.ds(c * LANES, LANES))
            rows_vmem[sl] = rows_vmem[sl] * g


def _gex_embed(ids_flat, gex_flat, table):
    mesh = plsc.VectorSubcoreMesh(core_axis_name="c", subcore_axis_name="s")
    cp = pltpu.CompilerParams()
    for _field, _val in (("needs_layout_passes", False),
                         ("use_tc_tiling_on_sc", False)):
        if _field in pltpu.CompilerParams.__dataclass_fields__:
            cp = dataclasses.replace(cp, **{_field: _val})

    @functools.partial(
        pl.kernel,
        out_type=jax.ShapeDtypeStruct(
            (LANDMARK_GENES, BATCH, HIDDEN_SIZE), jnp.float32
        ),
        mesh=mesh,
        compiler_params=cp,
        scratch_types=[
            pltpu.VMEM((PER_WORKER,), jnp.int32),
            pltpu.VMEM((PER_WORKER,), jnp.float32),
            pltpu.VMEM((N_SLOTS * SUB, HIDDEN_SIZE), jnp.float32),
            pltpu.SemaphoreType.DMA,
            pltpu.SemaphoreType.DMA,
            pltpu.SemaphoreType.DMA((N_SLOTS,)),
            pltpu.SemaphoreType.DMA((N_SLOTS,)),
        ],
    )
    def k(table_hbm, ids_hbm, gex_hbm, out_hbm, ids_v, gex_v, rows_v,
          sem_ids, sem_gex, gsem, osem):
        wid = lax.axis_index("s") * 2 + lax.axis_index("c")
        row0 = wid * PER_WORKER

        stage_i = pltpu.async_copy(
            ids_hbm.at[pl.ds(row0, PER_WORKER)], ids_v, sem_ids)
        stage_g = pltpu.async_copy(
            gex_hbm.at[pl.ds(row0, PER_WORKER)], gex_v, sem_gex)
        stage_i.wait()
        stage_g.wait()

        def slot_rows(slot):
            return rows_v.at[pl.ds(slot * SUB, SUB)]

        @pl.loop(0, N_STEPS + LEAD)
        def _(s):
            @pl.when(s < N_STEPS)
            def _():
                slot = lax.rem(s, N_SLOTS)

                @pl.when(s >= N_SLOTS)
                def _():
                    pltpu.make_async_copy(
                        slot_rows(slot), out_hbm.at[0, pl.ds(0, SUB)],
                        osem.at[slot]).wait()

                pltpu.async_copy(
                    table_hbm.at[ids_v.at[pl.ds(s * SUB, SUB)]],
                    slot_rows(slot), gsem.at[slot])

            t = s - LEAD

            @pl.when(t >= 0)
            def _():
                slot2 = lax.rem(t, N_SLOTS)
                pltpu.make_async_copy(
                    table_hbm.at[ids_v.at[pl.ds(0, SUB)]],
                    slot_rows(slot2), gsem.at[slot2]).wait()
                _scale_rows(rows_v, gex_v, slot2 * SUB, t * SUB)
                row = row0 + t * SUB
                gene = row // BATCH
                off = lax.rem(row, BATCH)
                pltpu.async_copy(
                    slot_rows(slot2), out_hbm.at[gene, pl.ds(off, SUB)],
                    osem.at[slot2])

        @pl.loop(0, N_SLOTS)
        def _(q):
            pltpu.make_async_copy(
                slot_rows(q), out_hbm.at[0, pl.ds(0, SUB)],
                osem.at[q]).wait()

    return k(table, ids_flat, gex_flat)


def kernel(gene_expression, gene_input_ids, bool_masked_pos, group_mtx, gene_embedding_table):
    del bool_masked_pos, group_mtx
    ids_flat = gene_input_ids.astype(jnp.int32).T.reshape(-1)
    gex_flat = gene_expression.astype(jnp.float32).T.reshape(-1)
    out_t = _gex_embed(ids_flat, gex_flat, gene_embedding_table)
    return jnp.transpose(out_t, (1, 0, 2))


# speedup vs baseline: 1.2300x; 1.0025x over previous
import dataclasses
import functools

import jax
import jax.numpy as jnp
from jax import lax
from jax.experimental import pallas as pl
from jax.experimental.pallas import tpu as pltpu
from jax.experimental.pallas import tpu_sc as plsc

LANDMARK_GENES = 978
VOCAB_SIZE = 20000
HIDDEN_SIZE = 128
BATCH = 1024

LANES = 16
N_WORKERS = 32
SUB = 64
N_SLOTS = 8
LEAD = 6
N_ROWS = LANDMARK_GENES * BATCH
N_STEPS = N_ROWS // (SUB * N_WORKERS)
PER_WORKER = N_STEPS * SUB


def _scale_rows(rows_vmem, gex_vmem, row_lo, gex_lo):
    off = gex_lo - row_lo

    @plsc.parallel_loop(row_lo, row_lo + SUB, unroll=16)
    def _(r):
        gidx = jnp.full((LANES,), r + off, jnp.int32)
        g = plsc.load_gather(gex_vmem, [gidx])
        for c in range(HIDDEN_SIZE // LANES):
            sl = (r, pl.ds(c * LANES, LANES))
            rows_vmem[sl] = rows_vmem[sl] * g


def _gex_embed(ids_flat, gex_flat, table):
    mesh = plsc.VectorSubcoreMesh(core_axis_name="c", subcore_axis_name="s")
    cp = pltpu.CompilerParams()
    for _field, _val in (("needs_layout_passes", False),
                         ("use_tc_tiling_on_sc", False)):
        if _field in pltpu.CompilerParams.__dataclass_fields__:
            cp = dataclasses.replace(cp, **{_field: _val})

    @functools.partial(
        pl.kernel,
        out_type=jax.ShapeDtypeStruct(
            (LANDMARK_GENES, BATCH, HIDDEN_SIZE), jnp.float32
        ),
        mesh=mesh,
        compiler_params=cp,
        scratch_types=[
            pltpu.VMEM((PER_WORKER,), jnp.int32),
            pltpu.VMEM((PER_WORKER,), jnp.float32),
            pltpu.VMEM((N_SLOTS * SUB, HIDDEN_SIZE), jnp.float32),
            pltpu.SemaphoreType.DMA,
            pltpu.SemaphoreType.DMA,
            pltpu.SemaphoreType.DMA((N_SLOTS,)),
            pltpu.SemaphoreType.DMA((N_SLOTS,)),
        ],
    )
    def k(table_hbm, ids_hbm, gex_hbm, out_hbm, ids_v, gex_v, rows_v,
          sem_ids, sem_gex, gsem, osem):
        wid = lax.axis_index("s") * 2 + lax.axis_index("c")
        row0 = wid * PER_WORKER

        stage_i = pltpu.async_copy(
            ids_hbm.at[pl.ds(row0, PER_WORKER)], ids_v, sem_ids)
        stage_g = pltpu.async_copy(
            gex_hbm.at[pl.ds(row0, PER_WORKER)], gex_v, sem_gex)
        stage_i.wait()
        stage_g.wait()

        def slot_rows(slot):
            return rows_v.at[pl.ds(slot * SUB, SUB)]

        @pl.loop(0, N_STEPS + LEAD)
        def _(s):
            @pl.when(s < N_STEPS)
            def _():
                slot = lax.rem(s, N_SLOTS)

                @pl.when(s >= N_SLOTS)
                def _():
                    pltpu.make_async_copy(
                        slot_rows(slot), out_hbm.at[0, pl.ds(0, SUB)],
                        osem.at[slot]).wait()

                pltpu.async_copy(
                    table_hbm.at[ids_v.at[pl.ds(s * SUB, SUB)]],
                    slot_rows(slot), gsem.at[slot])

            t = s - LEAD

            @pl.when(t >= 0)
            def _():
                slot2 = lax.rem(t, N_SLOTS)
                pltpu.make_async_copy(
                    table_hbm.at[ids_v.at[pl.ds(0, SUB)]],
                    slot_rows(slot2), gsem.at[slot2]).wait()
                _scale_rows(rows_v, gex_v, slot2 * SUB, t * SUB)
                row = row0 + t * SUB
                gene = row // BATCH
                off = lax.rem(row, BATCH)
                pltpu.async_copy(
                    slot_rows(slot2), out_hbm.at[gene, pl.ds(off, SUB)],
                    osem.at[slot2])

        @pl.loop(0, N_SLOTS)
        def _(q):
            pltpu.make_async_copy(
                slot_rows(q), out_hbm.at[0, pl.ds(0, SUB)],
                osem.at[q]).wait()

    return k(table, ids_flat, gex_flat)


def kernel(gene_expression, gene_input_ids, bool_masked_pos, group_mtx, gene_embedding_table):
    del bool_masked_pos, group_mtx
    ids_flat = gene_input_ids.astype(jnp.int32).T.reshape(-1)
    gex_flat = gene_expression.astype(jnp.float32).T.reshape(-1)
    out_t = _gex_embed(ids_flat, gex_flat, gene_embedding_table)
    return jnp.transpose(out_t, (1, 0, 2))


# speedup vs baseline: 1.2398x; 1.0080x over previous
import dataclasses
import functools

import jax
import jax.numpy as jnp
from jax import lax
from jax.experimental import pallas as pl
from jax.experimental.pallas import tpu as pltpu
from jax.experimental.pallas import tpu_sc as plsc

LANDMARK_GENES = 978
VOCAB_SIZE = 20000
HIDDEN_SIZE = 128
BATCH = 1024

LANES = 16
N_WORKERS = 32
SUB = 128
N_SLOTS = 4
LEAD = 2
N_ROWS = LANDMARK_GENES * BATCH
N_STEPS = N_ROWS // (SUB * N_WORKERS)
PER_WORKER = N_STEPS * SUB


def _scale_rows(rows_vmem, gex_vmem, row_lo, gex_lo):
    off = gex_lo - row_lo

    @plsc.parallel_loop(row_lo, row_lo + SUB, unroll=16)
    def _(r):
        gidx = jnp.full((LANES,), r + off, jnp.int32)
        g = plsc.load_gather(gex_vmem, [gidx])
        for c in range(HIDDEN_SIZE // LANES):
            sl = (r, pl.ds(c * LANES, LANES))
            rows_vmem[sl] = rows_vmem[sl] * g


def _gex_embed(ids_flat, gex_flat, table):
    mesh = plsc.VectorSubcoreMesh(core_axis_name="c", subcore_axis_name="s")
    cp = pltpu.CompilerParams()
    for _field, _val in (("needs_layout_passes", False),
                         ("use_tc_tiling_on_sc", False)):
        if _field in pltpu.CompilerParams.__dataclass_fields__:
            cp = dataclasses.replace(cp, **{_field: _val})

    @functools.partial(
        pl.kernel,
        out_type=jax.ShapeDtypeStruct(
            (LANDMARK_GENES, BATCH, HIDDEN_SIZE), jnp.float32
        ),
        mesh=mesh,
        compiler_params=cp,
        scratch_types=[
            pltpu.VMEM((PER_WORKER,), jnp.int32),
            pltpu.VMEM((PER_WORKER,), jnp.float32),
            pltpu.VMEM((N_SLOTS * SUB, HIDDEN_SIZE), jnp.float32),
            pltpu.SemaphoreType.DMA,
            pltpu.SemaphoreType.DMA,
            pltpu.SemaphoreType.DMA((N_SLOTS,)),
            pltpu.SemaphoreType.DMA((N_SLOTS,)),
        ],
    )
    def k(table_hbm, ids_hbm, gex_hbm, out_hbm, ids_v, gex_v, rows_v,
          sem_ids, sem_gex, gsem, osem):
        wid = lax.axis_index("s") * 2 + lax.axis_index("c")
        row0 = wid * PER_WORKER

        stage_i = pltpu.async_copy(
            ids_hbm.at[pl.ds(row0, PER_WORKER)], ids_v, sem_ids)
        stage_g = pltpu.async_copy(
            gex_hbm.at[pl.ds(row0, PER_WORKER)], gex_v, sem_gex)
        stage_i.wait()
        stage_g.wait()

        def slot_rows(slot):
            return rows_v.at[pl.ds(slot * SUB, SUB)]

        @pl.loop(0, N_STEPS + LEAD)
        def _(s):
            @pl.when(s < N_STEPS)
            def _():
                slot = lax.rem(s, N_SLOTS)

                @pl.when(s >= N_SLOTS)
                def _():
                    pltpu.make_async_copy(
                        slot_rows(slot), out_hbm.at[0, pl.ds(0, SUB)],
                        osem.at[slot]).wait()

                pltpu.async_copy(
                    table_hbm.at[ids_v.at[pl.ds(s * SUB, SUB)]],
                    slot_rows(slot), gsem.at[slot])

            t = s - LEAD

            @pl.when(t >= 0)
            def _():
                slot2 = lax.rem(t, N_SLOTS)
                pltpu.make_async_copy(
                    table_hbm.at[ids_v.at[pl.ds(0, SUB)]],
                    slot_rows(slot2), gsem.at[slot2]).wait()
                _scale_rows(rows_v, gex_v, slot2 * SUB, t * SUB)
                row = row0 + t * SUB
                gene = row // BATCH
                off = lax.rem(row, BATCH)
                pltpu.async_copy(
                    slot_rows(slot2), out_hbm.at[gene, pl.ds(off, SUB)],
                    osem.at[slot2])

        @pl.loop(0, N_SLOTS)
        def _(q):
            pltpu.make_async_copy(
                slot_rows(q), out_hbm.at[0, pl.ds(0, SUB)],
                osem.at[q]).wait()

    return k(table, ids_flat, gex_flat)


def kernel(gene_expression, gene_input_ids, bool_masked_pos, group_mtx, gene_embedding_table):
    del bool_masked_pos, group_mtx
    ids_flat = gene_input_ids.astype(jnp.int32).T.reshape(-1)
    gex_flat = gene_expression.astype(jnp.float32).T.reshape(-1)
    out_t = _gex_embed(ids_flat, gex_flat, gene_embedding_table)
    return jnp.transpose(out_t, (1, 0, 2))
